# baseline (speedup 1.0000x reference)
import functools

import jax
import jax.numpy as jnp
from jax import lax
from jax.experimental import pallas as pl
from jax.experimental.pallas import tpu as pltpu
from jax.experimental.pallas import tpu_sc as plsc

_LANES = 16
_NC = 2
_NS = 16
_NW = _NC * _NS
_CP = 48
_CH = 80


def _transpose_tc(w48):
    cp, n = w48.shape

    def body(in_ref, out_ref):
        out_ref[...] = in_ref[...].T

    return pl.pallas_call(
        body,
        out_shape=jax.ShapeDtypeStruct((n, cp), w48.dtype),
    )(w48)


def _sc_accumulate(ei, wt):
    cpt = ei.shape[2]
    n, cp = wt.shape
    rpt = (n // _NS) // 8 * 8
    rem = n - _NS * rpt

    mesh = plsc.VectorSubcoreMesh(core_axis_name="c", subcore_axis_name="s")

    @functools.partial(
        pl.kernel,
        mesh=mesh,
        out_type=(
            jax.ShapeDtypeStruct((_NC, 2 * n + 16, cp), jnp.float32),
            jax.ShapeDtypeStruct((_NW, _LANES), jnp.int32),
        ),
        scratch_types=[
            pltpu.VMEM_SHARED((n, cp), jnp.float32),
            pltpu.VMEM((cpt, _CH), jnp.int32),
            pltpu.VMEM((cpt, _CH), jnp.int32),
            [pltpu.VMEM((_CH, cp), jnp.float32)] * 12,
            pltpu.VMEM((rpt // 8, cp), jnp.float32),
            pltpu.VMEM((_LANES,), jnp.int32),
            [pltpu.SemaphoreType.DMA] * 12,
            [pltpu.SemaphoreType.DMA] * 12,
        ],
        compiler_params=pltpu.CompilerParams(use_tc_tiling_on_sc=False),
    )
    def k1(ei_ref, wt_ref, acc_ref, min_ref, acc_s, colb, rowb, msgs, zbuf,
           minv, gsems, ssems):
        c = lax.axis_index("c")
        s = lax.axis_index("s")
        tid = c * _NS + s

        nbuf = 12
        k = 8

        def gather(j, b):
            pltpu.async_copy(wt_ref.at[colb.at[j]], msgs[b], gsems[b])

        def swait(b):
            pltpu.make_async_copy(msgs[b], acc_s.at[rowb.at[0]],
                                  ssems[b]).wait()

        pltpu.sync_copy(ei_ref.at[1, tid], colb)
        pltpu.sync_copy(ei_ref.at[0, tid], rowb)
        for j in range(k):
            gather(j, j)

        zero = jnp.zeros((_LANES,), jnp.float32)
        zrows = rpt // 8

        def zrow(r, carry):
            for u in range(cp // _LANES):
                zbuf[r, pl.ds(u * _LANES, _LANES)] = zero
            return carry

        lax.fori_loop(0, zrows, zrow, 0)
        for t in range(8):
            pltpu.async_copy(
                zbuf, acc_s.at[pl.ds(s * rpt + t * zrows, zrows), :],
                ssems[t])
        for t in range(8):
            pltpu.make_async_copy(
                zbuf, acc_s.at[pl.ds(s * rpt + t * zrows, zrows), :],
                ssems[t]).wait()

        @pl.when(s == 0)
        def _zero_tail():
            pltpu.sync_copy(zbuf.at[pl.ds(0, rem), :],
                            acc_s.at[pl.ds(_NS * rpt, rem), :])

        plsc.subcore_barrier()

        def step(i, b, mm):
            bn = (b + k) % nbuf
            pltpu.make_async_copy(
                wt_ref.at[colb.at[i]], msgs[b], gsems[b]).wait()
            pltpu.async_copy(msgs[b], acc_s.at[rowb.at[i]], ssems[b],
                             add=True)

            @pl.when(i >= nbuf - k)
            def _settle():
                swait(bn)

            @pl.when(i + k < cpt)
            def _refill():
                gather(i + k, bn)

            for u in range(_CH // _LANES):
                mm = jnp.minimum(mm, rowb[i, pl.ds(u * _LANES, _LANES)])
            return mm

        def group(g, mm):
            for b in range(nbuf):
                mm = step(g * nbuf + b, b, mm)
            return mm

        mm = lax.fori_loop(
            0, cpt // nbuf, group,
            jnp.full((_LANES,), jnp.iinfo(jnp.int32).max, jnp.int32))
        for i in range(cpt - cpt % nbuf, cpt):
            mm = step(i, i % nbuf, mm)
        for t in range(nbuf - k):
            swait((cpt - (nbuf - k) + t) % nbuf)

        minv[...] = mm
        pltpu.sync_copy(minv, min_ref.at[tid])

        plsc.subcore_barrier()
        pltpu.sync_copy(acc_s.at[pl.ds(s * rpt, rpt), :],
                        acc_ref.at[c, pl.ds(s * rpt, rpt), :])

        @pl.when(s == 0)
        def _publish_tail():
            pltpu.sync_copy(acc_s.at[pl.ds(_NS * rpt, rem), :],
                            acc_ref.at[c, pl.ds(_NS * rpt, rem), :])

    return k1(ei, wt)


def _sc_combine(acc, mins, b, n, co):
    cp = _CP
    nrt = -(-n // _NW)
    nlast = n - nrt * (_NW - 1)

    mesh = plsc.VectorSubcoreMesh(core_axis_name="c", subcore_axis_name="s")

    @functools.partial(
        pl.kernel,
        mesh=mesh,
        out_type=jax.ShapeDtypeStruct((n, co), jnp.float32),
        scratch_types=[
            pltpu.VMEM((nrt, cp), jnp.float32),
            pltpu.VMEM((nrt, cp), jnp.float32),
            pltpu.VMEM((nrt, cp), jnp.float32),
            pltpu.VMEM((cp,), jnp.float32),
            pltpu.VMEM((_NW, _LANES), jnp.int32),
            [pltpu.SemaphoreType.DMA] * 2,
        ],
        compiler_params=pltpu.CompilerParams(use_tc_tiling_on_sc=False),
    )
    def k2(acc_ref, min_ref, b_ref, out_ref, a0, a1, ob, bb, mb, asems):
        c = lax.axis_index("c")
        s = lax.axis_index("s")
        tid = c * _NS + s

        pltpu.sync_copy(min_ref, mb)
        pltpu.sync_copy(b_ref, bb.at[pl.ds(0, co)])

        def mrow(i, mm):
            return jnp.minimum(mm, mb[i, :])

        mm = lax.fori_loop(
            0, _NW, mrow,
            jnp.full((_LANES,), jnp.iinfo(jnp.int32).max, jnp.int32))
        m = mm[0]
        for j in range(1, _LANES):
            m = jnp.minimum(m, mm[j])

        r0 = tid * nrt
        d0 = pltpu.async_copy(acc_ref.at[0, pl.ds(m + r0, nrt), :], a0,
                              asems[0])
        d1 = pltpu.async_copy(acc_ref.at[1, pl.ds(m + r0, nrt), :], a1,
                              asems[1])
        d0.wait()
        d1.wait()

        nvalid = n - m

        def row(g, carry):
            valid = (r0 + g) < nvalid
            for u in range(cp // _LANES):
                o = u * _LANES
                v = a0[g, pl.ds(o, _LANES)] + a1[g, pl.ds(o, _LANES)]
                v = jnp.where(valid, v, jnp.zeros_like(v))
                ob[g, pl.ds(o, _LANES)] = v + bb[pl.ds(o, _LANES)]
            return carry

        lax.fori_loop(0, nrt, row, 0)

        @pl.when(tid < _NW - 1)
        def _store():
            pltpu.sync_copy(ob.at[:, pl.ds(0, co)],
                            out_ref.at[pl.ds(r0, nrt), :])

        @pl.when(tid == _NW - 1)
        def _store_last():
            pltpu.sync_copy(ob.at[pl.ds(0, nlast), pl.ds(0, co)],
                            out_ref.at[pl.ds(r0, nlast), :])

    return k2(acc, mins, b)


def kernel(x, edge_index, W, b):
    del x
    c, n = W.shape
    e = edge_index.shape[1]

    w48 = jnp.concatenate(
        [W, jnp.zeros((_CP - c, n), W.dtype)], axis=0)
    ei = edge_index.reshape(2, _NW, e // (_NW * _CH), _CH)

    wt = w48.T
    acc, mins = _sc_accumulate(ei, wt)
    return _sc_combine(acc, mins, b, n, c)


# speedup vs baseline: 1.0178x; 1.0178x over previous
import functools

import jax
import jax.numpy as jnp
from jax import lax
from jax.experimental import pallas as pl
from jax.experimental.pallas import tpu as pltpu
from jax.experimental.pallas import tpu_sc as plsc

_LANES = 16
_NC = 2
_NS = 16
_NW = _NC * _NS
_CP = 48
_CH = 80


def _transpose_tc(w48):
    cp, n = w48.shape

    def body(in_ref, out_ref):
        out_ref[...] = in_ref[...].T

    return pl.pallas_call(
        body,
        out_shape=jax.ShapeDtypeStruct((n, cp), w48.dtype),
    )(w48)


def _sc_accumulate(ei, wt):
    cpt = ei.shape[2]
    n, cp = wt.shape
    rpt = (n // _NS) // 8 * 8
    rem = n - _NS * rpt

    mesh = plsc.VectorSubcoreMesh(core_axis_name="c", subcore_axis_name="s")

    @functools.partial(
        pl.kernel,
        mesh=mesh,
        out_type=(
            jax.ShapeDtypeStruct((_NC, 2 * n + 16, cp), jnp.float32),
            jax.ShapeDtypeStruct((_NW, _LANES), jnp.int32),
        ),
        scratch_types=[
            pltpu.VMEM_SHARED((n, cp), jnp.float32),
            pltpu.VMEM((cpt, _CH), jnp.int32),
            pltpu.VMEM((cpt, _CH), jnp.int32),
            [pltpu.VMEM((_CH, cp), jnp.float32)] * 10,
            pltpu.VMEM((rpt // 8, cp), jnp.float32),
            pltpu.VMEM((_LANES,), jnp.int32),
            [pltpu.SemaphoreType.DMA] * 10,
            [pltpu.SemaphoreType.DMA] * 10,
        ],
        compiler_params=pltpu.CompilerParams(use_tc_tiling_on_sc=False),
    )
    def k1(ei_ref, wt_ref, acc_ref, min_ref, acc_s, colb, rowb, msgs, zbuf,
           minv, gsems, ssems):
        c = lax.axis_index("c")
        s = lax.axis_index("s")
        tid = c * _NS + s

        nbuf = 10
        k = 7

        def gather(j, b):
            pltpu.async_copy(wt_ref.at[colb.at[j]], msgs[b], gsems[b])

        def swait(b):
            pltpu.make_async_copy(msgs[b], acc_s.at[rowb.at[0]],
                                  ssems[b]).wait()

        pltpu.sync_copy(ei_ref.at[1, tid], colb)
        pltpu.sync_copy(ei_ref.at[0, tid], rowb)
        for j in range(k):
            gather(j, j)

        zero = jnp.zeros((_LANES,), jnp.float32)
        zrows = rpt // 8

        def zrow(r, carry):
            for u in range(cp // _LANES):
                zbuf[r, pl.ds(u * _LANES, _LANES)] = zero
            return carry

        lax.fori_loop(0, zrows, zrow, 0)
        for t in range(8):
            pltpu.async_copy(
                zbuf, acc_s.at[pl.ds(s * rpt + t * zrows, zrows), :],
                ssems[t])
        for t in range(8):
            pltpu.make_async_copy(
                zbuf, acc_s.at[pl.ds(s * rpt + t * zrows, zrows), :],
                ssems[t]).wait()

        @pl.when(s == 0)
        def _zero_tail():
            pltpu.sync_copy(zbuf.at[pl.ds(0, rem), :],
                            acc_s.at[pl.ds(_NS * rpt, rem), :])

        plsc.subcore_barrier()

        def step(i, b, mm):
            bn = (b + k) % nbuf
            pltpu.make_async_copy(
                wt_ref.at[colb.at[i]], msgs[b], gsems[b]).wait()
            pltpu.async_copy(msgs[b], acc_s.at[rowb.at[i]], ssems[b],
                             add=True)

            @pl.when(i >= nbuf - k)
            def _settle():
                swait(bn)

            @pl.when(i + k < cpt)
            def _refill():
                gather(i + k, bn)

            for u in range(_CH // _LANES):
                mm = jnp.minimum(mm, rowb[i, pl.ds(u * _LANES, _LANES)])
            return mm

        def group(g, mm):
            for b in range(nbuf):
                mm = step(g * nbuf + b, b, mm)
            return mm

        mm = lax.fori_loop(
            0, cpt // nbuf, group,
            jnp.full((_LANES,), jnp.iinfo(jnp.int32).max, jnp.int32))
        for i in range(cpt - cpt % nbuf, cpt):
            mm = step(i, i % nbuf, mm)
        for t in range(nbuf - k):
            swait((cpt - (nbuf - k) + t) % nbuf)

        minv[...] = mm
        pltpu.sync_copy(minv, min_ref.at[tid])

        plsc.subcore_barrier()
        pltpu.sync_copy(acc_s.at[pl.ds(s * rpt, rpt), :],
                        acc_ref.at[c, pl.ds(s * rpt, rpt), :])

        @pl.when(s == 0)
        def _publish_tail():
            pltpu.sync_copy(acc_s.at[pl.ds(_NS * rpt, rem), :],
                            acc_ref.at[c, pl.ds(_NS * rpt, rem), :])

    return k1(ei, wt)


def _sc_combine(acc, mins, b, n, co):
    cp = _CP
    nrt = -(-n // _NW)
    nlast = n - nrt * (_NW - 1)

    mesh = plsc.VectorSubcoreMesh(core_axis_name="c", subcore_axis_name="s")

    @functools.partial(
        pl.kernel,
        mesh=mesh,
        out_type=jax.ShapeDtypeStruct((n, co), jnp.float32),
        scratch_types=[
            pltpu.VMEM((nrt, cp), jnp.float32),
            pltpu.VMEM((nrt, cp), jnp.float32),
            pltpu.VMEM((nrt, cp), jnp.float32),
            pltpu.VMEM((cp,), jnp.float32),
            pltpu.VMEM((_NW, _LANES), jnp.int32),
            [pltpu.SemaphoreType.DMA] * 2,
        ],
        compiler_params=pltpu.CompilerParams(use_tc_tiling_on_sc=False),
    )
    def k2(acc_ref, min_ref, b_ref, out_ref, a0, a1, ob, bb, mb, asems):
        c = lax.axis_index("c")
        s = lax.axis_index("s")
        tid = c * _NS + s

        pltpu.sync_copy(min_ref, mb)
        pltpu.sync_copy(b_ref, bb.at[pl.ds(0, co)])

        def mrow(i, mm):
            return jnp.minimum(mm, mb[i, :])

        mm = lax.fori_loop(
            0, _NW, mrow,
            jnp.full((_LANES,), jnp.iinfo(jnp.int32).max, jnp.int32))
        m = mm[0]
        for j in range(1, _LANES):
            m = jnp.minimum(m, mm[j])

        r0 = tid * nrt
        d0 = pltpu.async_copy(acc_ref.at[0, pl.ds(m + r0, nrt), :], a0,
                              asems[0])
        d1 = pltpu.async_copy(acc_ref.at[1, pl.ds(m + r0, nrt), :], a1,
                              asems[1])
        d0.wait()
        d1.wait()

        nvalid = n - m

        def row(g, carry):
            valid = (r0 + g) < nvalid
            for u in range(cp // _LANES):
                o = u * _LANES
                v = a0[g, pl.ds(o, _LANES)] + a1[g, pl.ds(o, _LANES)]
                v = jnp.where(valid, v, jnp.zeros_like(v))
                ob[g, pl.ds(o, _LANES)] = v + bb[pl.ds(o, _LANES)]
            return carry

        lax.fori_loop(0, nrt, row, 0)

        @pl.when(tid < _NW - 1)
        def _store():
            pltpu.sync_copy(ob.at[:, pl.ds(0, co)],
                            out_ref.at[pl.ds(r0, nrt), :])

        @pl.when(tid == _NW - 1)
        def _store_last():
            pltpu.sync_copy(ob.at[pl.ds(0, nlast), pl.ds(0, co)],
                            out_ref.at[pl.ds(r0, nlast), :])

    return k2(acc, mins, b)


def kernel(x, edge_index, W, b):
    del x
    c, n = W.shape
    e = edge_index.shape[1]

    w48 = jnp.concatenate(
        [W, jnp.zeros((_CP - c, n), W.dtype)], axis=0)
    ei = edge_index.reshape(2, _NW, e // (_NW * _CH), _CH)

    wt = w48.T
    acc, mins = _sc_accumulate(ei, wt)
    return _sc_combine(acc, mins, b, n, c)
